# baseline (speedup 1.0000x reference)
import functools

import jax
import jax.numpy as jnp
from jax import lax
from jax.experimental import pallas as pl
from jax.experimental.pallas import tpu as pltpu
from jax.experimental.pallas import tpu_sc as plsc

N = 10000
D = 128
NC = 2
NS = 16
NW = NC * NS
B = 128
G = 8
N_PAD = 10240
ROWS = N_PAD // NS


def _make_aggregate(groups, with_counts):
    mesh = plsc.VectorSubcoreMesh(core_axis_name="c", subcore_axis_name="s")

    out_type = [jax.ShapeDtypeStruct((NC, N_PAD, D), jnp.float32)]
    if with_counts:
        out_type.append(jax.ShapeDtypeStruct((NC, N_PAD, D), jnp.float32))
    scratch = [
        pltpu.VMEM_SHARED((N_PAD, D), jnp.float32),
        pltpu.VMEM((G, B), jnp.int32),
        pltpu.VMEM((G, B), jnp.int32),
        pltpu.VMEM((B, D), jnp.float32),
        pltpu.VMEM((B, D), jnp.float32),
        pltpu.SemaphoreType.DMA,
        pltpu.SemaphoreType.DMA,
    ]

    def body(feats, srci, dsti, zf, ones_in, *refs):
        if with_counts:
            psum, pcnt, acc, srcg, dstg, rows0, rows1, sem0, sem1 = refs
        else:
            psum, acc, srcg, dstg, rows0, rows1, sem0, sem1 = refs
        rows = (rows0, rows1)
        sems = (sem0, sem1)
        c = lax.axis_index("c")
        s = lax.axis_index("s")
        wid = c * NS + s

        if with_counts:
            pltpu.sync_copy(zf, acc.at[pl.ds(s * ROWS, ROWS)])
            pltpu.sync_copy(ones_in, rows0)
            plsc.subcore_barrier()

            def cstep(g, carry):
                pltpu.sync_copy(dsti.at[wid].at[pl.ds(g * G, G)], dstg)
                for j in range(G):
                    pltpu.sync_copy(rows0, acc.at[dstg.at[j]], add=True)
                return carry

            lax.fori_loop(0, groups, cstep, 0)
            plsc.subcore_barrier()
            pltpu.sync_copy(acc.at[pl.ds(s * ROWS, ROWS)],
                            pcnt.at[c].at[pl.ds(s * ROWS, ROWS)])
            plsc.subcore_barrier()

        pltpu.sync_copy(zf, acc.at[pl.ds(s * ROWS, ROWS)])
        plsc.subcore_barrier()

        def step(g, carry):
            pltpu.sync_copy(srci.at[wid].at[pl.ds(g * G, G)], srcg)
            pltpu.sync_copy(dsti.at[wid].at[pl.ds(g * G, G)], dstg)
            pltpu.async_copy(feats.at[srcg.at[0]], rows0, sem0)
            for j in range(G):
                if j + 1 < G:
                    pltpu.async_copy(feats.at[srcg.at[j + 1]],
                                     rows[(j + 1) % 2], sems[(j + 1) % 2])
                pltpu.make_async_copy(feats.at[srcg.at[j]], rows[j % 2],
                                      sems[j % 2]).wait()
                pltpu.sync_copy(rows[j % 2], acc.at[dstg.at[j]], add=True)
            return carry

        lax.fori_loop(0, groups, step, 0)
        plsc.subcore_barrier()
        pltpu.sync_copy(acc.at[pl.ds(s * ROWS, ROWS)],
                        psum.at[c].at[pl.ds(s * ROWS, ROWS)])

    return pl.kernel(body, out_type=out_type, mesh=mesh,
                     scratch_types=scratch)


def _dense_body(act, p0r, p1r, c0r, c1r, xr, wlr, blr, wrr, outr):
    cnt = c0r[...][:, 0:1] + c1r[...][:, 0:1]
    inv = 1.0 / jnp.maximum(cnt, 1.0)
    mean = (p0r[...] + p1r[...]) * inv
    y = (jnp.dot(mean, wlr[...], preferred_element_type=jnp.float32)
         + jnp.dot(xr[...], wrr[...], preferred_element_type=jnp.float32)
         + blr[...])
    if act:
        y = jnp.where(y > 0.0, y, jnp.exp(jnp.minimum(y, 0.0)) - 1.0)
    outr[...] = y


def _dense(p0, p1, c0, c1, x, Wl, bl, Wr, act):
    bn = 1000
    grid = (N // bn,)
    row_spec = pl.BlockSpec((bn, D), lambda i: (i, 0))
    w_spec = pl.BlockSpec((D, D), lambda i: (0, 0))
    b_spec = pl.BlockSpec((1, D), lambda i: (0, 0))
    return pl.pallas_call(
        functools.partial(_dense_body, act),
        grid=grid,
        in_specs=[row_spec, row_spec, row_spec, row_spec, row_spec,
                  w_spec, b_spec, w_spec],
        out_specs=row_spec,
        out_shape=jax.ShapeDtypeStruct((N, D), jnp.float32),
    )(p0, p1, c0, c1, x, Wl, bl.reshape(1, D), Wr)


def kernel(x, edge_index, W1l, b1l, W1r, W2l, b2l, W2r):
    src = edge_index[0]
    dst = edge_index[1]
    e = src.shape[0]
    chunks = -(-e // (NW * B))
    groups = -(-chunks // G)
    chunks = groups * G
    pad = chunks * NW * B - e
    if pad:
        src = jnp.concatenate([src, jnp.zeros((pad,), jnp.int32)])
        dst = jnp.concatenate([dst, jnp.full((pad,), N, jnp.int32)])
    src3 = src.reshape(NW, chunks, B)
    dst3 = dst.reshape(NW, chunks, B)
    zf = jnp.zeros((ROWS, D), jnp.float32)
    ones = jnp.ones((B, D), jnp.float32)

    agg1 = _make_aggregate(groups, with_counts=True)
    agg2 = _make_aggregate(groups, with_counts=False)

    psum, pcnt = agg1(x, src3, dst3, zf, ones)
    c0 = pcnt[0, :N]
    c1 = pcnt[1, :N]
    h = _dense(psum[0, :N], psum[1, :N], c0, c1, x, W1l, b1l, W1r, act=True)
    psum2 = agg2(h, src3, dst3, zf, ones)
    if isinstance(psum2, (list, tuple)):
        psum2 = psum2[0]
    return _dense(psum2[0, :N], psum2[1, :N], c0, c1, h, W2l, b2l, W2r,
                  act=False)
